# Initial kernel scaffold; baseline (speedup 1.0000x reference)
import jax
import jax.numpy as jnp
from jax.experimental import pallas as pl


def kernel(pos, batch, w1, b1, g1, be1, rm1, rv1, w2, b2, g2, be2, rm2, rv2, w3, b3, w4, b4, wl, bl, wm1, bm1, wm2, bm2, wm3, bm3):
    raise NotImplementedError("write your pallas kernel here")


# speedup vs baseline: 19.7538x; 19.7538x over previous
import functools

import jax
import jax.numpy as jnp
from jax import lax
from jax.experimental import pallas as pl
from jax.experimental.pallas import tpu as pltpu
from jax.experimental.pallas import tpu_sc as plsc

_B = 16
_K = 20
_P = 768
_KW = 32


def _dot(a, b):
    return lax.dot_general(a, b, (((1,), (0,)), ((), ())),
                           preferred_element_type=jnp.float32)


def _knn_body(starts_ref, x_ref, idx_ref, d_ref, *, P, K, KW):
    s = pl.program_id(0)
    st = starts_ref[s]
    size = starts_ref[s + 1] - st
    x = x_ref[pl.ds(st, P), :]
    sq = jnp.sum(x * x, axis=1, keepdims=True)
    g = lax.dot_general(x, x, (((1,), (1,)), ((), ())),
                        preferred_element_type=jnp.float32)
    d2 = (sq - 2.0 * g) + jnp.transpose(sq)
    col = lax.broadcasted_iota(jnp.int32, (P, P), 1)
    BIG = jnp.float32(3e38)
    d_ref[...] = jnp.where(col < size, d2, BIG)
    kcol = lax.broadcasted_iota(jnp.int32, (P, KW), 1)

    def body(k, loc):
        d = d_ref[...]
        m = jnp.min(d, axis=1, keepdims=True)
        a = jnp.min(jnp.where(d <= m, col, P), axis=1, keepdims=True)
        d_ref[...] = jnp.where(col == a, BIG, d)
        return jnp.where(kcol == k, a, loc)

    loc = lax.fori_loop(0, K, body, jnp.zeros((P, KW), jnp.int32))
    idx_ref[pl.ds(st, P), :] = loc + st


def _knn(xp, starts, *, F):
    npad = xp.shape[0]
    grid_spec = pltpu.PrefetchScalarGridSpec(
        num_scalar_prefetch=1,
        grid=(_B,),
        in_specs=[pl.BlockSpec((npad, F), lambda s, st: (0, 0))],
        out_specs=pl.BlockSpec((npad, _KW), lambda s, st: (0, 0)),
        scratch_shapes=[pltpu.VMEM((_P, _P), jnp.float32)],
    )
    return pl.pallas_call(
        functools.partial(_knn_body, P=_P, K=_K, KW=_KW),
        grid_spec=grid_spec,
        out_shape=jax.ShapeDtypeStruct((npad, _KW), jnp.int32),
    )(starts, xp)


def _sc_gather(table, idx, *, D, CH):
    nb = idx.shape[0]
    NW = 32
    bpw = nb // NW
    mesh = plsc.VectorSubcoreMesh(core_axis_name="c", subcore_axis_name="s")

    @functools.partial(
        pl.kernel, mesh=mesh,
        out_type=jax.ShapeDtypeStruct((nb, D), jnp.float32),
        scratch_types=[pltpu.VMEM((bpw,), jnp.int32),
                       pltpu.VMEM((CH, D), jnp.float32),
                       pltpu.SemaphoreType.DMA],
    )
    def gk(table_hbm, idx_hbm, out_hbm, idx_v, rows_v, sem):
        wid = lax.axis_index("s") * 2 + lax.axis_index("c")
        base = wid * bpw
        pltpu.sync_copy(idx_hbm.at[pl.ds(base, bpw)], idx_v)
        for c in range(bpw // CH):
            pltpu.async_copy(
                table_hbm.at[idx_v.at[pl.ds(c * CH, CH)]], rows_v, sem).wait()
            pltpu.sync_copy(rows_v, out_hbm.at[pl.ds(base + c * CH, CH)])

    return gk(table, idx)


def _edge1_body(u_ref, g_ref, w1_ref, b1_ref, rm1_ref, q1_ref, ga1_ref,
                be1_ref, w2_ref, b2_ref, rm2_ref, q2_ref, ga2_ref, be2_ref,
                w3_ref, b3_ref, x1_ref, *, R, K):
    u = u_ref[...]
    m = jnp.concatenate(
        [u + g_ref[:, k * 128:k * 128 + 8] for k in range(K)], axis=0)
    z = _dot(m, w1_ref[...]) + b1_ref[...]
    h = jnp.maximum((z - rm1_ref[...]) / q1_ref[...] * ga1_ref[...]
                    + be1_ref[...], 0.0)
    z = _dot(h, w2_ref[...]) + b2_ref[...]
    h = jnp.maximum((z - rm2_ref[...]) / q2_ref[...] * ga2_ref[...]
                    + be2_ref[...], 0.0)
    e = _dot(h, w3_ref[...]) + b3_ref[...]
    x1 = e[0:R]
    for k in range(1, K):
        x1 = jnp.maximum(x1, e[k * R:(k + 1) * R])
    x1_ref[...] = x1


def _edge1(u1, gj, params, *, R):
    n = u1.shape[0]
    row = lambda v: v[None, :]
    (w1p, b1, rm1, q1, ga1, be1, w2, b2, rm2, q2, ga2, be2, w3, b3) = params
    ins = (u1, gj, w1p, row(b1), row(rm1), row(q1), row(ga1), row(be1),
           w2, row(b2), row(rm2), row(q2), row(ga2), row(be2), w3, row(b3))
    specs = [pl.BlockSpec((R, 8), lambda i: (i, 0)),
             pl.BlockSpec((R, _K * 128), lambda i: (i, 0)),
             pl.BlockSpec((8, 64), lambda i: (0, 0))]
    specs += [pl.BlockSpec((1, 64), lambda i: (0, 0))] * 5
    specs += [pl.BlockSpec((64, 64), lambda i: (0, 0))]
    specs += [pl.BlockSpec((1, 64), lambda i: (0, 0))] * 5
    specs += [pl.BlockSpec((64, 64), lambda i: (0, 0)),
              pl.BlockSpec((1, 64), lambda i: (0, 0))]
    return pl.pallas_call(
        functools.partial(_edge1_body, R=R, K=_K),
        grid=(n // R,),
        in_specs=specs,
        out_specs=pl.BlockSpec((R, 64), lambda i: (i, 0)),
        out_shape=jax.ShapeDtypeStruct((n, 64), jnp.float32),
    )(*ins)


def _pool_body(starts_ref, x1_ref, u2_ref, g2_ref, w4_ref, b4_ref,
               wl_ref, bl_ref, out_ref, *, R, K, B):
    pid = pl.program_id(0)
    u2 = u2_ref[...]
    m2 = jnp.concatenate(
        [u2 + g2_ref[:, k * 128:(k + 1) * 128] for k in range(K)], axis=0)
    e = _dot(m2, w4_ref[...]) + b4_ref[...]
    x2 = e[0:R]
    for k in range(1, K):
        x2 = jnp.maximum(x2, e[k * R:(k + 1) * R])
    feat = jnp.concatenate([x1_ref[...], x2], axis=1)
    lin = _dot(feat, wl_ref[...]) + bl_ref[...]
    rows = pid * R + lax.broadcasted_iota(jnp.int32, (R, 1), 0)
    NEG = jnp.float32(-3e38)
    parts = []
    for s in range(B):
        msk = (rows >= starts_ref[s]) & (rows < starts_ref[s + 1])
        parts.append(jnp.max(jnp.where(msk, lin, NEG), axis=0, keepdims=True))
    cur = jnp.concatenate(parts, axis=0)

    @pl.when(pid == 0)
    def _():
        out_ref[...] = jnp.full(out_ref.shape, NEG, jnp.float32)

    out_ref[...] = jnp.maximum(out_ref[...], cur)


def _pool(x1, u2, g2v, w4, b4, wl, bl, starts, *, R):
    n = x1.shape[0]
    grid_spec = pltpu.PrefetchScalarGridSpec(
        num_scalar_prefetch=1,
        grid=(n // R,),
        in_specs=[pl.BlockSpec((R, 64), lambda i, st: (i, 0)),
                  pl.BlockSpec((R, 128), lambda i, st: (i, 0)),
                  pl.BlockSpec((R, _K * 128), lambda i, st: (i, 0)),
                  pl.BlockSpec((128, 128), lambda i, st: (0, 0)),
                  pl.BlockSpec((1, 128), lambda i, st: (0, 0)),
                  pl.BlockSpec((192, 1024), lambda i, st: (0, 0)),
                  pl.BlockSpec((1, 1024), lambda i, st: (0, 0))],
        out_specs=pl.BlockSpec((_B, 1024), lambda i, st: (0, 0)),
    )
    return pl.pallas_call(
        functools.partial(_pool_body, R=R, K=_K, B=_B),
        grid_spec=grid_spec,
        out_shape=jax.ShapeDtypeStruct((_B, 1024), jnp.float32),
    )(starts, x1, u2, g2v, w4, b4[None, :], wl, bl[None, :])


def _head_body(p_ref, w1_ref, b1_ref, w2_ref, b2_ref, w3_ref, b3_ref, o_ref):
    h = jnp.maximum(_dot(p_ref[...], w1_ref[...]) + b1_ref[...], 0.0)
    h = jnp.maximum(_dot(h, w2_ref[...]) + b2_ref[...], 0.0)
    o_ref[...] = _dot(h, w3_ref[...]) + b3_ref[...]


def _head(pooled, wm1, bm1, wm2, bm2, wm3, bm3):
    return pl.pallas_call(
        _head_body,
        out_shape=jax.ShapeDtypeStruct((_B, 40), jnp.float32),
    )(pooled, wm1, bm1[None, :], wm2, bm2[None, :], wm3, bm3[None, :])


def kernel(pos, batch, w1, b1, g1, be1, rm1, rv1, w2, b2, g2, be2, rm2, rv2,
           w3, b3, w4, b4, wl, bl, wm1, bm1, wm2, bm2, wm3, bm3):
    n = pos.shape[0]
    npad = n + _P
    starts = jnp.searchsorted(
        batch, jnp.arange(_B + 1, dtype=jnp.int32), side='left').astype(jnp.int32)

    q1 = jnp.sqrt(rv1 + 1e-5)
    q2 = jnp.sqrt(rv2 + 1e-5)
    w1p = jnp.pad(w1, ((0, 2), (0, 0)))
    zc = jnp.zeros((n, 2), jnp.float32)
    u1 = jnp.concatenate([pos, -pos, zc], axis=1)
    tbl1 = jnp.pad(pos, ((0, 0), (3, 122)))

    posp = jnp.pad(pos, ((0, npad - n), (0, 5)))
    idx1 = _knn(posp, starts, F=8)[:n, :_K]
    gj = _sc_gather(tbl1, idx1.reshape(-1), D=128, CH=512)
    x1 = _edge1(u1, gj.reshape(n, _K * 128),
                (w1p, b1, rm1, q1, g1, be1, w2, b2, rm2, q2, g2, be2, w3, b3),
                R=512)

    u2 = jnp.concatenate([x1, -x1], axis=1)
    tbl2 = jnp.pad(x1, ((0, 0), (64, 0)))
    x1p = jnp.pad(x1, ((0, npad - n), (0, 0)))
    idx2 = _knn(x1p, starts, F=64)[:n, :_K]
    g2r = _sc_gather(tbl2, idx2.reshape(-1), D=128, CH=512)

    pooled = _pool(x1, u2, g2r.reshape(n, _K * 128), w4, b4, wl, bl,
                   starts, R=256)
    return _head(pooled, wm1, bm1, wm2, bm2, wm3, bm3)


# speedup vs baseline: 22.7640x; 1.1524x over previous
import functools

import jax
import jax.numpy as jnp
from jax import lax
from jax.experimental import pallas as pl
from jax.experimental.pallas import tpu as pltpu
from jax.experimental.pallas import tpu_sc as plsc

_B = 16
_K = 20
_P = 640


def _dot(a, b):
    return lax.dot_general(a, b, (((1,), (0,)), ((), ())),
                           preferred_element_type=jnp.float32)


def _knn_body(starts_ref, x_ref, idx_ref, d_ref, *, P, K):
    s = pl.program_id(0)
    st = starts_ref[s]
    size = starts_ref[s + 1] - st
    x = x_ref[pl.ds(st, P), :]
    sq = jnp.sum(x * x, axis=1, keepdims=True)
    g = lax.dot_general(x, x, (((1,), (1,)), ((), ())),
                        preferred_element_type=jnp.float32)
    d2 = (sq - 2.0 * g) + jnp.transpose(sq)
    col = lax.broadcasted_iota(jnp.int32, (P, P), 1)
    BIG = jnp.float32(3e38)
    d_ref[...] = jnp.where(col < size, d2, BIG)
    kcol = lax.broadcasted_iota(jnp.int32, (P, K), 1)

    def body(k, loc):
        d = d_ref[...]
        m = jnp.min(d, axis=1, keepdims=True)
        a = jnp.min(jnp.where(d <= m, col, P), axis=1, keepdims=True)
        d_ref[...] = jnp.where(col == a, BIG, d)
        return jnp.where(kcol == k, a, loc)

    loc = lax.fori_loop(0, K, body, jnp.zeros((P, K), jnp.int32))
    idx_ref[pl.ds(st, P), :] = loc + st


def _knn(xp, starts, *, F):
    npad = xp.shape[0]
    grid_spec = pltpu.PrefetchScalarGridSpec(
        num_scalar_prefetch=1,
        grid=(_B,),
        in_specs=[pl.BlockSpec((npad, F), lambda s, st: (0, 0))],
        out_specs=pl.BlockSpec((npad, _K), lambda s, st: (0, 0)),
        scratch_shapes=[pltpu.VMEM((_P, _P), jnp.float32)],
    )
    return pl.pallas_call(
        functools.partial(_knn_body, P=_P, K=_K),
        grid_spec=grid_spec,
        out_shape=jax.ShapeDtypeStruct((npad, _K), jnp.int32),
    )(starts, xp)


def _sc_gather(table, idx, *, nb, D, CH):
    NW = 32
    bpw = nb // NW
    mesh = plsc.VectorSubcoreMesh(core_axis_name="c", subcore_axis_name="s")

    @functools.partial(
        pl.kernel, mesh=mesh,
        out_type=jax.ShapeDtypeStruct((nb, D), jnp.float32),
        scratch_types=[pltpu.VMEM((bpw,), jnp.int32),
                       pltpu.VMEM((CH, D), jnp.float32),
                       pltpu.SemaphoreType.DMA],
    )
    def gk(table_hbm, idx_hbm, out_hbm, idx_v, rows_v, sem):
        wid = lax.axis_index("s") * 2 + lax.axis_index("c")
        base = wid * bpw
        pltpu.sync_copy(idx_hbm.at[pl.ds(base, bpw)], idx_v)
        for c in range(bpw // CH):
            pltpu.async_copy(
                table_hbm.at[idx_v.at[pl.ds(c * CH, CH)]], rows_v, sem).wait()
            pltpu.sync_copy(rows_v, out_hbm.at[pl.ds(base + c * CH, CH)])

    return gk(table, idx)


def _edge1_body(u_ref, g_ref, w1_ref, b1_ref, rm1_ref, q1_ref, ga1_ref,
                be1_ref, w2_ref, b2_ref, rm2_ref, q2_ref, ga2_ref, be2_ref,
                w3_ref, b3_ref, x1_ref, u2_ref, t2_ref, *, R, K):
    u = u_ref[...]
    m = jnp.concatenate(
        [u + g_ref[:, k * 128:k * 128 + 8] for k in range(K)], axis=0)
    z = _dot(m, w1_ref[...]) + b1_ref[...]
    h = jnp.maximum((z - rm1_ref[...]) / q1_ref[...] * ga1_ref[...]
                    + be1_ref[...], 0.0)
    z = _dot(h, w2_ref[...]) + b2_ref[...]
    h = jnp.maximum((z - rm2_ref[...]) / q2_ref[...] * ga2_ref[...]
                    + be2_ref[...], 0.0)
    e = _dot(h, w3_ref[...]) + b3_ref[...]
    x1 = e[0:R]
    for k in range(1, K):
        x1 = jnp.maximum(x1, e[k * R:(k + 1) * R])
    x1_ref[...] = x1
    u2_ref[...] = jnp.concatenate([x1, -x1], axis=1)
    t2_ref[...] = jnp.concatenate([jnp.zeros((R, 64), jnp.float32), x1],
                                  axis=1)


def _edge1(u1, gj, params, *, R, npad):
    n = u1.shape[0]
    row = lambda v: v[None, :]
    (w1p, b1, rm1, q1, ga1, be1, w2, b2, rm2, q2, ga2, be2, w3, b3) = params
    ins = (u1, gj, w1p, row(b1), row(rm1), row(q1), row(ga1), row(be1),
           w2, row(b2), row(rm2), row(q2), row(ga2), row(be2), w3, row(b3))
    specs = [pl.BlockSpec((R, 8), lambda i: (i, 0)),
             pl.BlockSpec((R, _K * 128), lambda i: (i, 0)),
             pl.BlockSpec((8, 64), lambda i: (0, 0))]
    specs += [pl.BlockSpec((1, 64), lambda i: (0, 0))] * 5
    specs += [pl.BlockSpec((64, 64), lambda i: (0, 0))]
    specs += [pl.BlockSpec((1, 64), lambda i: (0, 0))] * 5
    specs += [pl.BlockSpec((64, 64), lambda i: (0, 0)),
              pl.BlockSpec((1, 64), lambda i: (0, 0))]
    return pl.pallas_call(
        functools.partial(_edge1_body, R=R, K=_K),
        grid=(n // R,),
        in_specs=specs,
        out_specs=[pl.BlockSpec((R, 64), lambda i: (i, 0)),
                   pl.BlockSpec((R, 128), lambda i: (i, 0)),
                   pl.BlockSpec((R, 128), lambda i: (i, 0))],
        out_shape=[jax.ShapeDtypeStruct((npad, 64), jnp.float32),
                   jax.ShapeDtypeStruct((n, 128), jnp.float32),
                   jax.ShapeDtypeStruct((n, 128), jnp.float32)],
    )(*ins)


def _pool_body(starts_ref, x1_ref, u2_ref, g2_ref, w4_ref, b4_ref,
               wl_ref, bl_ref, out_ref, *, R, K, B):
    pid = pl.program_id(0)
    u2 = u2_ref[...]
    m2 = jnp.concatenate(
        [u2 + g2_ref[:, k * 128:(k + 1) * 128] for k in range(K)], axis=0)
    e = _dot(m2, w4_ref[...]) + b4_ref[...]
    x2 = e[0:R]
    for k in range(1, K):
        x2 = jnp.maximum(x2, e[k * R:(k + 1) * R])
    feat = jnp.concatenate([x1_ref[...], x2], axis=1)
    lin = _dot(feat, wl_ref[...]) + bl_ref[...]
    rows = pid * R + lax.broadcasted_iota(jnp.int32, (R, 1), 0)
    NEG = jnp.float32(-3e38)
    parts = []
    for s in range(B):
        msk = (rows >= starts_ref[s]) & (rows < starts_ref[s + 1])
        parts.append(jnp.max(jnp.where(msk, lin, NEG), axis=0, keepdims=True))
    cur = jnp.concatenate(parts, axis=0)

    @pl.when(pid == 0)
    def _():
        out_ref[...] = jnp.full(out_ref.shape, NEG, jnp.float32)

    out_ref[...] = jnp.maximum(out_ref[...], cur)


def _pool(x1, u2, g2v, w4, b4, wl, bl, starts, *, R):
    n = u2.shape[0]
    grid_spec = pltpu.PrefetchScalarGridSpec(
        num_scalar_prefetch=1,
        grid=(n // R,),
        in_specs=[pl.BlockSpec((R, 64), lambda i, st: (i, 0)),
                  pl.BlockSpec((R, 128), lambda i, st: (i, 0)),
                  pl.BlockSpec((R, _K * 128), lambda i, st: (i, 0)),
                  pl.BlockSpec((128, 128), lambda i, st: (0, 0)),
                  pl.BlockSpec((1, 128), lambda i, st: (0, 0)),
                  pl.BlockSpec((192, 1024), lambda i, st: (0, 0)),
                  pl.BlockSpec((1, 1024), lambda i, st: (0, 0))],
        out_specs=pl.BlockSpec((_B, 1024), lambda i, st: (0, 0)),
    )
    return pl.pallas_call(
        functools.partial(_pool_body, R=R, K=_K, B=_B),
        grid_spec=grid_spec,
        out_shape=jax.ShapeDtypeStruct((_B, 1024), jnp.float32),
    )(starts, x1, u2, g2v, w4, b4[None, :], wl, bl[None, :])


def _head_body(p_ref, w1_ref, b1_ref, w2_ref, b2_ref, w3_ref, b3_ref, o_ref):
    h = jnp.maximum(_dot(p_ref[...], w1_ref[...]) + b1_ref[...], 0.0)
    h = jnp.maximum(_dot(h, w2_ref[...]) + b2_ref[...], 0.0)
    o_ref[...] = _dot(h, w3_ref[...]) + b3_ref[...]


def _head(pooled, wm1, bm1, wm2, bm2, wm3, bm3):
    return pl.pallas_call(
        _head_body,
        out_shape=jax.ShapeDtypeStruct((_B, 40), jnp.float32),
    )(pooled, wm1, bm1[None, :], wm2, bm2[None, :], wm3, bm3[None, :])


def kernel(pos, batch, w1, b1, g1, be1, rm1, rv1, w2, b2, g2, be2, rm2, rv2,
           w3, b3, w4, b4, wl, bl, wm1, bm1, wm2, bm2, wm3, bm3):
    n = pos.shape[0]
    npad = n + _P
    starts = jnp.searchsorted(
        batch, jnp.arange(_B + 1, dtype=jnp.int32), side='left').astype(jnp.int32)

    q1 = jnp.sqrt(rv1 + 1e-5)
    q2 = jnp.sqrt(rv2 + 1e-5)
    w1p = jnp.pad(w1, ((0, 2), (0, 0)))
    zc = jnp.zeros((n, 2), jnp.float32)
    u1 = jnp.concatenate([pos, -pos, zc], axis=1)
    tbl1 = jnp.pad(pos, ((0, 0), (3, 122)))

    posp = jnp.pad(pos, ((0, npad - n), (0, 5)))
    idx1 = _knn(posp, starts, F=8)
    gj = _sc_gather(tbl1, idx1.reshape(-1), nb=n * _K, D=128, CH=512)
    x1p, u2, tbl2 = _edge1(
        u1, gj.reshape(n, _K * 128),
        (w1p, b1, rm1, q1, g1, be1, w2, b2, rm2, q2, g2, be2, w3, b3),
        R=512, npad=npad)

    idx2 = _knn(x1p, starts, F=64)
    g2r = _sc_gather(tbl2, idx2.reshape(-1), nb=n * _K, D=128, CH=512)

    pooled = _pool(x1p, u2, g2r.reshape(n, _K * 128), w4, b4, wl, bl,
                   starts, R=256)
    return _head(pooled, wm1, bm1, wm2, bm2, wm3, bm3)


# speedup vs baseline: 28.9543x; 1.2719x over previous
import functools

import jax
import jax.numpy as jnp
from jax import lax
from jax.experimental import pallas as pl
from jax.experimental.pallas import tpu as pltpu
from jax.experimental.pallas import tpu_sc as plsc

_B = 16
_K = 20
_P = 640


def _dot(a, b):
    return lax.dot_general(a, b, (((1,), (0,)), ((), ())),
                           preferred_element_type=jnp.float32)


def _knn_body(starts_ref, x_ref, idx_ref, d_ref, *, P, K):
    s = pl.program_id(0)
    st = starts_ref[s]
    size = starts_ref[s + 1] - st
    x = x_ref[pl.ds(st, P), :]
    sq = jnp.sum(x * x, axis=1, keepdims=True)
    g = lax.dot_general(x, x, (((1,), (1,)), ((), ())),
                        preferred_element_type=jnp.float32)
    d2 = (sq - 2.0 * g) + jnp.transpose(sq)
    col = lax.broadcasted_iota(jnp.int32, (P, P), 1)
    BIG = jnp.float32(3e38)
    d_ref[...] = jnp.where(col < size, d2, BIG)
    kcol = lax.broadcasted_iota(jnp.int32, (P, K), 1)

    def body(k, loc):
        d = d_ref[...]
        m = jnp.min(d, axis=1, keepdims=True)
        a = jnp.min(jnp.where(d <= m, col, P), axis=1, keepdims=True)
        d_ref[...] = jnp.where(col == a, BIG, d)
        return jnp.where(kcol == k, a, loc)

    loc = lax.fori_loop(0, K, body, jnp.zeros((P, K), jnp.int32))
    idx_ref[pl.ds(st, P), :] = loc + st


def _knn(xp, starts, *, F):
    npad = xp.shape[0]
    grid_spec = pltpu.PrefetchScalarGridSpec(
        num_scalar_prefetch=1,
        grid=(_B,),
        in_specs=[pl.BlockSpec((npad, F), lambda s, st: (0, 0))],
        out_specs=pl.BlockSpec((npad, _K), lambda s, st: (0, 0)),
        scratch_shapes=[pltpu.VMEM((_P, _P), jnp.float32)],
    )
    return pl.pallas_call(
        functools.partial(_knn_body, P=_P, K=_K),
        grid_spec=grid_spec,
        out_shape=jax.ShapeDtypeStruct((npad, _K), jnp.int32),
    )(starts, xp)


def _sc_gather(table, idx, *, nb, D, CH):
    NW = 32
    bpw = nb // NW
    mesh = plsc.VectorSubcoreMesh(core_axis_name="c", subcore_axis_name="s")

    @functools.partial(
        pl.kernel, mesh=mesh,
        out_type=jax.ShapeDtypeStruct((nb, D), jnp.float32),
        scratch_types=[pltpu.VMEM((bpw,), jnp.int32),
                       pltpu.VMEM((CH, D), jnp.float32),
                       pltpu.SemaphoreType.DMA],
        compiler_params=pltpu.CompilerParams(use_tc_tiling_on_sc=False),
    )
    def gk(table_hbm, idx_hbm, out_hbm, idx_v, rows_v, sem):
        wid = lax.axis_index("s") * 2 + lax.axis_index("c")
        base = wid * bpw
        pltpu.sync_copy(idx_hbm.at[pl.ds(base, bpw)], idx_v)
        for c in range(bpw // CH):
            pltpu.async_copy(
                table_hbm.at[idx_v.at[pl.ds(c * CH, CH)]], rows_v, sem).wait()
            pltpu.sync_copy(rows_v, out_hbm.at[pl.ds(base + c * CH, CH)])

    return gk(table, idx)


def _edge1_body(u_ref, g_ref, w1_ref, b1_ref, rm1_ref, q1_ref, ga1_ref,
                be1_ref, w2_ref, b2_ref, rm2_ref, q2_ref, ga2_ref, be2_ref,
                w3_ref, b3_ref, x1_ref, *, R, K):
    u = u_ref[...]
    m = jnp.concatenate(
        [u + g_ref[:, k * 8:(k + 1) * 8] for k in range(K)], axis=0)
    z = _dot(m, w1_ref[...]) + b1_ref[...]
    h = jnp.maximum((z - rm1_ref[...]) / q1_ref[...] * ga1_ref[...]
                    + be1_ref[...], 0.0)
    z = _dot(h, w2_ref[...]) + b2_ref[...]
    h = jnp.maximum((z - rm2_ref[...]) / q2_ref[...] * ga2_ref[...]
                    + be2_ref[...], 0.0)
    e = _dot(h, w3_ref[...]) + b3_ref[...]
    x1 = e[0:R]
    for k in range(1, K):
        x1 = jnp.maximum(x1, e[k * R:(k + 1) * R])
    x1_ref[...] = x1


def _edge1(u1, gj, params, *, R, npad):
    n = u1.shape[0]
    row = lambda v: v[None, :]
    (w1p, b1, rm1, q1, ga1, be1, w2, b2, rm2, q2, ga2, be2, w3, b3) = params
    ins = (u1, gj, w1p, row(b1), row(rm1), row(q1), row(ga1), row(be1),
           w2, row(b2), row(rm2), row(q2), row(ga2), row(be2), w3, row(b3))
    specs = [pl.BlockSpec((R, 8), lambda i: (i, 0)),
             pl.BlockSpec((R, _K * 8), lambda i: (i, 0)),
             pl.BlockSpec((8, 64), lambda i: (0, 0))]
    specs += [pl.BlockSpec((1, 64), lambda i: (0, 0))] * 5
    specs += [pl.BlockSpec((64, 64), lambda i: (0, 0))]
    specs += [pl.BlockSpec((1, 64), lambda i: (0, 0))] * 5
    specs += [pl.BlockSpec((64, 64), lambda i: (0, 0)),
              pl.BlockSpec((1, 64), lambda i: (0, 0))]
    return pl.pallas_call(
        functools.partial(_edge1_body, R=R, K=_K),
        grid=(n // R,),
        in_specs=specs,
        out_specs=pl.BlockSpec((R, 64), lambda i: (i, 0)),
        out_shape=jax.ShapeDtypeStruct((npad, 64), jnp.float32),
    )(*ins)


def _pool_body(starts_ref, x1_ref, g2_ref, w4_ref, b4_ref,
               wl_ref, bl_ref, out_ref, *, R, K, B):
    pid = pl.program_id(0)
    x1 = x1_ref[...]
    m2 = jnp.concatenate(
        [jnp.concatenate([x1, g2_ref[:, k * 64:(k + 1) * 64] - x1], axis=1)
         for k in range(K)], axis=0)
    e = _dot(m2, w4_ref[...]) + b4_ref[...]
    x2 = e[0:R]
    for k in range(1, K):
        x2 = jnp.maximum(x2, e[k * R:(k + 1) * R])
    feat = jnp.concatenate([x1, x2], axis=1)
    lin = _dot(feat, wl_ref[...]) + bl_ref[...]
    rows = pid * R + lax.broadcasted_iota(jnp.int32, (R, 1), 0)
    NEG = jnp.float32(-3e38)
    parts = []
    for s in range(B):
        msk = (rows >= starts_ref[s]) & (rows < starts_ref[s + 1])
        parts.append(jnp.max(jnp.where(msk, lin, NEG), axis=0, keepdims=True))
    cur = jnp.concatenate(parts, axis=0)

    @pl.when(pid == 0)
    def _():
        out_ref[...] = jnp.full(out_ref.shape, NEG, jnp.float32)

    out_ref[...] = jnp.maximum(out_ref[...], cur)


def _pool(x1, g2v, w4, b4, wl, bl, starts, *, R, n):
    grid_spec = pltpu.PrefetchScalarGridSpec(
        num_scalar_prefetch=1,
        grid=(n // R,),
        in_specs=[pl.BlockSpec((R, 64), lambda i, st: (i, 0)),
                  pl.BlockSpec((R, _K * 64), lambda i, st: (i, 0)),
                  pl.BlockSpec((128, 128), lambda i, st: (0, 0)),
                  pl.BlockSpec((1, 128), lambda i, st: (0, 0)),
                  pl.BlockSpec((192, 1024), lambda i, st: (0, 0)),
                  pl.BlockSpec((1, 1024), lambda i, st: (0, 0))],
        out_specs=pl.BlockSpec((_B, 1024), lambda i, st: (0, 0)),
    )
    return pl.pallas_call(
        functools.partial(_pool_body, R=R, K=_K, B=_B),
        grid_spec=grid_spec,
        out_shape=jax.ShapeDtypeStruct((_B, 1024), jnp.float32),
    )(starts, x1, g2v, w4, b4[None, :], wl, bl[None, :])


def _head_body(p_ref, w1_ref, b1_ref, w2_ref, b2_ref, w3_ref, b3_ref, o_ref):
    h = jnp.maximum(_dot(p_ref[...], w1_ref[...]) + b1_ref[...], 0.0)
    h = jnp.maximum(_dot(h, w2_ref[...]) + b2_ref[...], 0.0)
    o_ref[...] = _dot(h, w3_ref[...]) + b3_ref[...]


def _head(pooled, wm1, bm1, wm2, bm2, wm3, bm3):
    return pl.pallas_call(
        _head_body,
        out_shape=jax.ShapeDtypeStruct((_B, 40), jnp.float32),
    )(pooled, wm1, bm1[None, :], wm2, bm2[None, :], wm3, bm3[None, :])


def kernel(pos, batch, w1, b1, g1, be1, rm1, rv1, w2, b2, g2, be2, rm2, rv2,
           w3, b3, w4, b4, wl, bl, wm1, bm1, wm2, bm2, wm3, bm3):
    n = pos.shape[0]
    npad = n + _P
    starts = jnp.searchsorted(
        batch, jnp.arange(_B + 1, dtype=jnp.int32), side='left').astype(jnp.int32)

    q1 = jnp.sqrt(rv1 + 1e-5)
    q2 = jnp.sqrt(rv2 + 1e-5)
    w1p = jnp.pad(w1, ((0, 2), (0, 0)))
    zc = jnp.zeros((n, 2), jnp.float32)
    u1 = jnp.concatenate([pos, -pos, zc], axis=1)
    tbl1 = jnp.pad(pos, ((0, 0), (3, 2)))

    posp = jnp.pad(pos, ((0, npad - n), (0, 5)))
    idx1 = _knn(posp, starts, F=8)
    gj = _sc_gather(tbl1, idx1.reshape(-1), nb=n * _K, D=8, CH=512)
    x1p = _edge1(
        u1, gj.reshape(n, _K * 8),
        (w1p, b1, rm1, q1, g1, be1, w2, b2, rm2, q2, g2, be2, w3, b3),
        R=512, npad=npad)

    idx2 = _knn(x1p, starts, F=64)
    g2r = _sc_gather(x1p, idx2.reshape(-1), nb=n * _K, D=64, CH=512)

    pooled = _pool(x1p, g2r.reshape(n, _K * 64), w4, b4, wl, bl,
                   starts, R=256, n=n)
    return _head(pooled, wm1, bm1, wm2, bm2, wm3, bm3)


# speedup vs baseline: 29.0289x; 1.0026x over previous
import functools

import jax
import jax.numpy as jnp
from jax import lax
from jax.experimental import pallas as pl
from jax.experimental.pallas import tpu as pltpu
from jax.experimental.pallas import tpu_sc as plsc

_B = 16
_K = 20
_P = 640


def _dot(a, b):
    return lax.dot_general(a, b, (((1,), (0,)), ((), ())),
                           preferred_element_type=jnp.float32)


def _knn_body(starts_ref, x_ref, idx_ref, d_ref, *, P, K):
    s = pl.program_id(0)
    st = starts_ref[s]
    size = starts_ref[s + 1] - st
    x = x_ref[pl.ds(st, P), :]
    sq = jnp.sum(x * x, axis=1, keepdims=True)
    g = lax.dot_general(x, x, (((1,), (1,)), ((), ())),
                        preferred_element_type=jnp.float32)
    d2 = (sq - 2.0 * g) + jnp.transpose(sq)
    col = lax.broadcasted_iota(jnp.int32, (P, P), 1)
    BIG = jnp.float32(3e38)
    d0 = jnp.where(col < size, d2, BIG)
    d_ref[...] = d0
    m0 = jnp.min(d0, axis=1, keepdims=True)
    kcol = lax.broadcasted_iota(jnp.int32, (P, K), 1)

    def body(k, carry):
        loc, m = carry
        d = d_ref[...]
        hit = d <= m
        a = jnp.min(jnp.where(hit, col, P), axis=1, keepdims=True)
        dn = jnp.where(hit, BIG, d)
        d_ref[...] = dn
        mn = jnp.min(dn, axis=1, keepdims=True)
        return jnp.where(kcol == k, a, loc), mn

    loc, _ = lax.fori_loop(0, K, body,
                           (jnp.zeros((P, K), jnp.int32), m0))
    idx_ref[pl.ds(st, P), :] = loc + st


def _knn(xp, starts, *, F):
    npad = xp.shape[0]
    grid_spec = pltpu.PrefetchScalarGridSpec(
        num_scalar_prefetch=1,
        grid=(_B,),
        in_specs=[pl.BlockSpec((npad, F), lambda s, st: (0, 0))],
        out_specs=pl.BlockSpec((npad, _K), lambda s, st: (0, 0)),
        scratch_shapes=[pltpu.VMEM((_P, _P), jnp.float32)],
    )
    return pl.pallas_call(
        functools.partial(_knn_body, P=_P, K=_K),
        grid_spec=grid_spec,
        out_shape=jax.ShapeDtypeStruct((npad, _K), jnp.int32),
    )(starts, xp)


def _sc_gather(table, idx, *, nb, D, CH):
    NW = 32
    bpw = nb // NW
    mesh = plsc.VectorSubcoreMesh(core_axis_name="c", subcore_axis_name="s")

    @functools.partial(
        pl.kernel, mesh=mesh,
        out_type=jax.ShapeDtypeStruct((nb, D), jnp.float32),
        scratch_types=[pltpu.VMEM((bpw,), jnp.int32),
                       pltpu.VMEM((CH, D), jnp.float32),
                       pltpu.SemaphoreType.DMA],
        compiler_params=pltpu.CompilerParams(use_tc_tiling_on_sc=False),
    )
    def gk(table_hbm, idx_hbm, out_hbm, idx_v, rows_v, sem):
        wid = lax.axis_index("s") * 2 + lax.axis_index("c")
        base = wid * bpw
        pltpu.sync_copy(idx_hbm.at[pl.ds(base, bpw)], idx_v)
        for c in range(bpw // CH):
            pltpu.async_copy(
                table_hbm.at[idx_v.at[pl.ds(c * CH, CH)]], rows_v, sem).wait()
            pltpu.sync_copy(rows_v, out_hbm.at[pl.ds(base + c * CH, CH)])

    return gk(table, idx)


def _edge1_body(u_ref, g_ref, w1_ref, b1_ref, rm1_ref, q1_ref, ga1_ref,
                be1_ref, w2_ref, b2_ref, rm2_ref, q2_ref, ga2_ref, be2_ref,
                w3_ref, b3_ref, x1_ref, *, R, K):
    u = u_ref[...]
    m = jnp.concatenate(
        [u + g_ref[:, k * 8:(k + 1) * 8] for k in range(K)], axis=0)
    z = _dot(m, w1_ref[...]) + b1_ref[...]
    h = jnp.maximum((z - rm1_ref[...]) / q1_ref[...] * ga1_ref[...]
                    + be1_ref[...], 0.0)
    z = _dot(h, w2_ref[...]) + b2_ref[...]
    h = jnp.maximum((z - rm2_ref[...]) / q2_ref[...] * ga2_ref[...]
                    + be2_ref[...], 0.0)
    e = _dot(h, w3_ref[...]) + b3_ref[...]
    x1 = e[0:R]
    for k in range(1, K):
        x1 = jnp.maximum(x1, e[k * R:(k + 1) * R])
    x1_ref[...] = x1


def _edge1(u1, gj, params, *, R, npad):
    n = u1.shape[0]
    row = lambda v: v[None, :]
    (w1p, b1, rm1, q1, ga1, be1, w2, b2, rm2, q2, ga2, be2, w3, b3) = params
    ins = (u1, gj, w1p, row(b1), row(rm1), row(q1), row(ga1), row(be1),
           w2, row(b2), row(rm2), row(q2), row(ga2), row(be2), w3, row(b3))
    specs = [pl.BlockSpec((R, 8), lambda i: (i, 0)),
             pl.BlockSpec((R, _K * 8), lambda i: (i, 0)),
             pl.BlockSpec((8, 64), lambda i: (0, 0))]
    specs += [pl.BlockSpec((1, 64), lambda i: (0, 0))] * 5
    specs += [pl.BlockSpec((64, 64), lambda i: (0, 0))]
    specs += [pl.BlockSpec((1, 64), lambda i: (0, 0))] * 5
    specs += [pl.BlockSpec((64, 64), lambda i: (0, 0)),
              pl.BlockSpec((1, 64), lambda i: (0, 0))]
    return pl.pallas_call(
        functools.partial(_edge1_body, R=R, K=_K),
        grid=(n // R,),
        in_specs=specs,
        out_specs=pl.BlockSpec((R, 64), lambda i: (i, 0)),
        out_shape=jax.ShapeDtypeStruct((npad, 64), jnp.float32),
    )(*ins)


def _pool_body(starts_ref, x1_ref, g2_ref, w4_ref, b4_ref,
               wl_ref, bl_ref, out_ref, *, R, K, B):
    pid = pl.program_id(0)
    x1 = x1_ref[...]
    m2 = jnp.concatenate(
        [jnp.concatenate([x1, g2_ref[:, k * 64:(k + 1) * 64] - x1], axis=1)
         for k in range(K)], axis=0)
    e = _dot(m2, w4_ref[...]) + b4_ref[...]
    x2 = e[0:R]
    for k in range(1, K):
        x2 = jnp.maximum(x2, e[k * R:(k + 1) * R])
    feat = jnp.concatenate([x1, x2], axis=1)
    lin = _dot(feat, wl_ref[...]) + bl_ref[...]
    base = pid * R
    rows = base + lax.broadcasted_iota(jnp.int32, (R, 1), 0)
    NEG = jnp.float32(-3e38)

    @pl.when(pid == 0)
    def _():
        out_ref[...] = jnp.full(out_ref.shape, NEG, jnp.float32)

    s_lo = jnp.int32(0)
    s_hi = jnp.int32(0)
    for s in range(B):
        s_lo += jnp.where(starts_ref[s + 1] <= base, 1, 0).astype(jnp.int32)
        s_hi += jnp.where(starts_ref[s] < base + R, 1, 0).astype(jnp.int32)

    def seg_body(s, _):
        msk = (rows >= starts_ref[s]) & (rows < starts_ref[s + 1])
        part = jnp.max(jnp.where(msk, lin, NEG), axis=0, keepdims=True)
        out_ref[pl.ds(s, 1), :] = jnp.maximum(out_ref[pl.ds(s, 1), :], part)
        return 0

    lax.fori_loop(s_lo, s_hi, seg_body, 0)


def _pool(x1, g2v, w4, b4, wl, bl, starts, *, R, n):
    grid_spec = pltpu.PrefetchScalarGridSpec(
        num_scalar_prefetch=1,
        grid=(n // R,),
        in_specs=[pl.BlockSpec((R, 64), lambda i, st: (i, 0)),
                  pl.BlockSpec((R, _K * 64), lambda i, st: (i, 0)),
                  pl.BlockSpec((128, 128), lambda i, st: (0, 0)),
                  pl.BlockSpec((1, 128), lambda i, st: (0, 0)),
                  pl.BlockSpec((192, 1024), lambda i, st: (0, 0)),
                  pl.BlockSpec((1, 1024), lambda i, st: (0, 0))],
        out_specs=pl.BlockSpec((_B, 1024), lambda i, st: (0, 0)),
    )
    return pl.pallas_call(
        functools.partial(_pool_body, R=R, K=_K, B=_B),
        grid_spec=grid_spec,
        out_shape=jax.ShapeDtypeStruct((_B, 1024), jnp.float32),
    )(starts, x1, g2v, w4, b4[None, :], wl, bl[None, :])


def _head_body(p_ref, w1_ref, b1_ref, w2_ref, b2_ref, w3_ref, b3_ref, o_ref):
    h = jnp.maximum(_dot(p_ref[...], w1_ref[...]) + b1_ref[...], 0.0)
    h = jnp.maximum(_dot(h, w2_ref[...]) + b2_ref[...], 0.0)
    o_ref[...] = _dot(h, w3_ref[...]) + b3_ref[...]


def _head(pooled, wm1, bm1, wm2, bm2, wm3, bm3):
    return pl.pallas_call(
        _head_body,
        out_shape=jax.ShapeDtypeStruct((_B, 40), jnp.float32),
    )(pooled, wm1, bm1[None, :], wm2, bm2[None, :], wm3, bm3[None, :])


def kernel(pos, batch, w1, b1, g1, be1, rm1, rv1, w2, b2, g2, be2, rm2, rv2,
           w3, b3, w4, b4, wl, bl, wm1, bm1, wm2, bm2, wm3, bm3):
    n = pos.shape[0]
    npad = n + _P
    starts = jnp.searchsorted(
        batch, jnp.arange(_B + 1, dtype=jnp.int32), side='left').astype(jnp.int32)

    q1 = jnp.sqrt(rv1 + 1e-5)
    q2 = jnp.sqrt(rv2 + 1e-5)
    w1p = jnp.pad(w1, ((0, 2), (0, 0)))
    zc = jnp.zeros((n, 2), jnp.float32)
    u1 = jnp.concatenate([pos, -pos, zc], axis=1)
    tbl1 = jnp.pad(pos, ((0, 0), (3, 2)))

    posp = jnp.pad(pos, ((0, npad - n), (0, 5)))
    idx1 = _knn(posp, starts, F=8)
    gj = _sc_gather(tbl1, idx1.reshape(-1), nb=n * _K, D=8, CH=512)
    x1p = _edge1(
        u1, gj.reshape(n, _K * 8),
        (w1p, b1, rm1, q1, g1, be1, w2, b2, rm2, q2, g2, be2, w3, b3),
        R=512, npad=npad)

    idx2 = _knn(x1p, starts, F=64)
    g2r = _sc_gather(x1p, idx2.reshape(-1), nb=n * _K, D=64, CH=512)

    pooled = _pool(x1p, g2r.reshape(n, _K * 64), w4, b4, wl, bl,
                   starts, R=256, n=n)
    return _head(pooled, wm1, bm1, wm2, bm2, wm3, bm3)


# speedup vs baseline: 30.8018x; 1.0611x over previous
import functools

import jax
import jax.numpy as jnp
from jax import lax
from jax.experimental import pallas as pl
from jax.experimental.pallas import tpu as pltpu
from jax.experimental.pallas import tpu_sc as plsc

_B = 16
_K = 20
_P = 640


def _dot(a, b):
    return lax.dot_general(a, b, (((1,), (0,)), ((), ())),
                           preferred_element_type=jnp.float32)


def _knn_body(starts_ref, x_ref, idx_ref, d_ref, *, P, K):
    s = pl.program_id(0)
    st = starts_ref[s]
    size = starts_ref[s + 1] - st
    x = x_ref[pl.ds(st, P), :]
    sq = jnp.sum(x * x, axis=1, keepdims=True)
    g = lax.dot_general(x, x, (((1,), (1,)), ((), ())),
                        preferred_element_type=jnp.float32)
    d2 = (sq - 2.0 * g) + jnp.transpose(sq)
    col = lax.broadcasted_iota(jnp.int32, (P, P), 1)
    BIG = jnp.float32(3e38)
    d_ref[...] = jnp.where(col < size, d2, BIG)
    kcol = lax.broadcasted_iota(jnp.int32, (P, K), 1)

    def body(k, loc):
        d = d_ref[...]
        m = jnp.min(d, axis=1, keepdims=True)
        a = jnp.min(jnp.where(d <= m, col, P), axis=1, keepdims=True)
        d_ref[...] = jnp.where(col == a, BIG, d)
        return jnp.where(kcol == k, a, loc)

    loc = lax.fori_loop(0, K, body, jnp.zeros((P, K), jnp.int32))
    idx_ref[pl.ds(st, P), :] = loc + st


def _knn(xp, starts, *, F):
    npad = xp.shape[0]
    grid_spec = pltpu.PrefetchScalarGridSpec(
        num_scalar_prefetch=1,
        grid=(_B,),
        in_specs=[pl.BlockSpec((npad, F), lambda s, st: (0, 0))],
        out_specs=pl.BlockSpec((npad, _K), lambda s, st: (0, 0)),
        scratch_shapes=[pltpu.VMEM((_P, _P), jnp.float32)],
    )
    return pl.pallas_call(
        functools.partial(_knn_body, P=_P, K=_K),
        grid_spec=grid_spec,
        out_shape=jax.ShapeDtypeStruct((npad, _K), jnp.int32),
    )(starts, xp)


def _sc_gather(table, idx, *, nb, D, CH):
    NW = 32
    bpw = nb // NW
    mesh = plsc.VectorSubcoreMesh(core_axis_name="c", subcore_axis_name="s")

    @functools.partial(
        pl.kernel, mesh=mesh,
        out_type=jax.ShapeDtypeStruct((nb, D), jnp.float32),
        scratch_types=[pltpu.VMEM((bpw,), jnp.int32),
                       pltpu.VMEM((CH, D), jnp.float32),
                       pltpu.SemaphoreType.DMA],
        compiler_params=pltpu.CompilerParams(use_tc_tiling_on_sc=False),
    )
    def gk(table_hbm, idx_hbm, out_hbm, idx_v, rows_v, sem):
        wid = lax.axis_index("s") * 2 + lax.axis_index("c")
        base = wid * bpw
        pltpu.sync_copy(idx_hbm.at[pl.ds(base, bpw)], idx_v)
        for c in range(bpw // CH):
            pltpu.async_copy(
                table_hbm.at[idx_v.at[pl.ds(c * CH, CH)]], rows_v, sem).wait()
            pltpu.sync_copy(rows_v, out_hbm.at[pl.ds(base + c * CH, CH)])

    return gk(table, idx)


def _edge1_body(u_ref, g_ref, w1_ref, b1_ref, rm1_ref, q1_ref, ga1_ref,
                be1_ref, w2_ref, b2_ref, rm2_ref, q2_ref, ga2_ref, be2_ref,
                w3_ref, b3_ref, x1_ref, *, R, K):
    u = u_ref[...]
    m = jnp.concatenate(
        [u + g_ref[:, k * 8:(k + 1) * 8] for k in range(K)], axis=0)
    z = _dot(m, w1_ref[...]) + b1_ref[...]
    h = jnp.maximum((z - rm1_ref[...]) / q1_ref[...] * ga1_ref[...]
                    + be1_ref[...], 0.0)
    z = _dot(h, w2_ref[...]) + b2_ref[...]
    h = jnp.maximum((z - rm2_ref[...]) / q2_ref[...] * ga2_ref[...]
                    + be2_ref[...], 0.0)
    e = _dot(h, w3_ref[...]) + b3_ref[...]
    x1 = e[0:R]
    for k in range(1, K):
        x1 = jnp.maximum(x1, e[k * R:(k + 1) * R])
    x1_ref[...] = x1


def _edge1(u1, gj, params, *, R, npad):
    n = u1.shape[0]
    row = lambda v: v[None, :]
    (w1p, b1, rm1, q1, ga1, be1, w2, b2, rm2, q2, ga2, be2, w3, b3) = params
    ins = (u1, gj, w1p, row(b1), row(rm1), row(q1), row(ga1), row(be1),
           w2, row(b2), row(rm2), row(q2), row(ga2), row(be2), w3, row(b3))
    specs = [pl.BlockSpec((R, 8), lambda i: (i, 0)),
             pl.BlockSpec((R, _K * 8), lambda i: (i, 0)),
             pl.BlockSpec((8, 64), lambda i: (0, 0))]
    specs += [pl.BlockSpec((1, 64), lambda i: (0, 0))] * 5
    specs += [pl.BlockSpec((64, 64), lambda i: (0, 0))]
    specs += [pl.BlockSpec((1, 64), lambda i: (0, 0))] * 5
    specs += [pl.BlockSpec((64, 64), lambda i: (0, 0)),
              pl.BlockSpec((1, 64), lambda i: (0, 0))]
    return pl.pallas_call(
        functools.partial(_edge1_body, R=R, K=_K),
        grid=(n // R,),
        in_specs=specs,
        out_specs=pl.BlockSpec((R, 64), lambda i: (i, 0)),
        out_shape=jax.ShapeDtypeStruct((npad, 64), jnp.float32),
    )(*ins)


def _pool_body(starts_ref, x1_ref, g2_ref, w4_ref, b4_ref,
               wl_ref, bl_ref, out_ref, *, R, K, B):
    pid = pl.program_id(0)
    x1 = x1_ref[...]
    m2 = jnp.concatenate(
        [jnp.concatenate([x1, g2_ref[:, k * 64:(k + 1) * 64] - x1], axis=1)
         for k in range(K)], axis=0)
    e = _dot(m2, w4_ref[...]) + b4_ref[...]
    x2 = e[0:R]
    for k in range(1, K):
        x2 = jnp.maximum(x2, e[k * R:(k + 1) * R])
    feat = jnp.concatenate([x1, x2], axis=1)
    lin = _dot(feat, wl_ref[...]) + bl_ref[...]
    base = pid * R
    rows = base + lax.broadcasted_iota(jnp.int32, (R, 1), 0)
    NEG = jnp.float32(-3e38)

    @pl.when(pid == 0)
    def _():
        out_ref[...] = jnp.full(out_ref.shape, NEG, jnp.float32)

    s_lo = jnp.int32(0)
    s_hi = jnp.int32(0)
    for s in range(B):
        s_lo += jnp.where(starts_ref[s + 1] <= base, 1, 0).astype(jnp.int32)
        s_hi += jnp.where(starts_ref[s] < base + R, 1, 0).astype(jnp.int32)

    def seg_body(s, _):
        msk = (rows >= starts_ref[s]) & (rows < starts_ref[s + 1])
        part = jnp.max(jnp.where(msk, lin, NEG), axis=0, keepdims=True)
        out_ref[pl.ds(s, 1), :] = jnp.maximum(out_ref[pl.ds(s, 1), :], part)
        return 0

    lax.fori_loop(s_lo, s_hi, seg_body, 0)


def _pool(x1, g2v, w4, b4, wl, bl, starts, *, R, n):
    grid_spec = pltpu.PrefetchScalarGridSpec(
        num_scalar_prefetch=1,
        grid=(n // R,),
        in_specs=[pl.BlockSpec((R, 64), lambda i, st: (i, 0)),
                  pl.BlockSpec((R, _K * 64), lambda i, st: (i, 0)),
                  pl.BlockSpec((128, 128), lambda i, st: (0, 0)),
                  pl.BlockSpec((1, 128), lambda i, st: (0, 0)),
                  pl.BlockSpec((192, 1024), lambda i, st: (0, 0)),
                  pl.BlockSpec((1, 1024), lambda i, st: (0, 0))],
        out_specs=pl.BlockSpec((_B, 1024), lambda i, st: (0, 0)),
    )
    return pl.pallas_call(
        functools.partial(_pool_body, R=R, K=_K, B=_B),
        grid_spec=grid_spec,
        out_shape=jax.ShapeDtypeStruct((_B, 1024), jnp.float32),
    )(starts, x1, g2v, w4, b4[None, :], wl, bl[None, :])


def _head_body(p_ref, w1_ref, b1_ref, w2_ref, b2_ref, w3_ref, b3_ref, o_ref):
    h = jnp.maximum(_dot(p_ref[...], w1_ref[...]) + b1_ref[...], 0.0)
    h = jnp.maximum(_dot(h, w2_ref[...]) + b2_ref[...], 0.0)
    o_ref[...] = _dot(h, w3_ref[...]) + b3_ref[...]


def _head(pooled, wm1, bm1, wm2, bm2, wm3, bm3):
    return pl.pallas_call(
        _head_body,
        out_shape=jax.ShapeDtypeStruct((_B, 40), jnp.float32),
    )(pooled, wm1, bm1[None, :], wm2, bm2[None, :], wm3, bm3[None, :])


def kernel(pos, batch, w1, b1, g1, be1, rm1, rv1, w2, b2, g2, be2, rm2, rv2,
           w3, b3, w4, b4, wl, bl, wm1, bm1, wm2, bm2, wm3, bm3):
    n = pos.shape[0]
    npad = n + _P
    starts = jnp.searchsorted(
        batch, jnp.arange(_B + 1, dtype=jnp.int32), side='left').astype(jnp.int32)

    q1 = jnp.sqrt(rv1 + 1e-5)
    q2 = jnp.sqrt(rv2 + 1e-5)
    w1p = jnp.pad(w1, ((0, 2), (0, 0)))
    zc = jnp.zeros((n, 2), jnp.float32)
    u1 = jnp.concatenate([pos, -pos, zc], axis=1)
    tbl1 = jnp.pad(pos, ((0, 0), (3, 2)))

    posp = jnp.pad(pos, ((0, npad - n), (0, 5)))
    idx1 = _knn(posp, starts, F=8)
    gj = _sc_gather(tbl1, idx1.reshape(-1), nb=n * _K, D=8, CH=512)
    x1p = _edge1(
        u1, gj.reshape(n, _K * 8),
        (w1p, b1, rm1, q1, g1, be1, w2, b2, rm2, q2, g2, be2, w3, b3),
        R=512, npad=npad)

    idx2 = _knn(x1p, starts, F=64)
    g2r = _sc_gather(x1p, idx2.reshape(-1), nb=n * _K, D=64, CH=512)

    pooled = _pool(x1p, g2r.reshape(n, _K * 64), w4, b4, wl, bl,
                   starts, R=256, n=n)
    return _head(pooled, wm1, bm1, wm2, bm2, wm3, bm3)


# speedup vs baseline: 31.2759x; 1.0154x over previous
import functools

import jax
import jax.numpy as jnp
from jax import lax
from jax.experimental import pallas as pl
from jax.experimental.pallas import tpu as pltpu
from jax.experimental.pallas import tpu_sc as plsc

_B = 16
_K = 20
_P = 640


def _dot(a, b):
    return lax.dot_general(a, b, (((1,), (0,)), ((), ())),
                           preferred_element_type=jnp.float32)


def _knn_body(starts_ref, x_ref, idx_ref, d_ref, *, P, K):
    s = pl.program_id(0)
    st = starts_ref[s]
    size = starts_ref[s + 1] - st
    x = x_ref[pl.ds(st, P), :]
    sq = jnp.sum(x * x, axis=1, keepdims=True)
    g = lax.dot_general(x, x, (((1,), (1,)), ((), ())),
                        preferred_element_type=jnp.float32)
    d2 = (sq - 2.0 * g) + jnp.transpose(sq)
    col = lax.broadcasted_iota(jnp.int32, (P, P), 1)
    BIG = jnp.float32(3e38)
    d_ref[...] = jnp.where(col < size, d2, BIG)
    kcol = lax.broadcasted_iota(jnp.int32, (P, K), 1)

    def body(k, loc):
        d = d_ref[...]
        m = jnp.min(d, axis=1, keepdims=True)
        a = jnp.min(jnp.where(d <= m, col, P), axis=1, keepdims=True)
        d_ref[...] = jnp.where(col == a, BIG, d)
        return jnp.where(kcol == k, a, loc)

    loc = lax.fori_loop(0, K, body, jnp.zeros((P, K), jnp.int32))
    idx_ref[pl.ds(st, P), :] = loc + st


def _knn(xp, starts, *, F):
    npad = xp.shape[0]
    grid_spec = pltpu.PrefetchScalarGridSpec(
        num_scalar_prefetch=1,
        grid=(_B,),
        in_specs=[pl.BlockSpec((npad, F), lambda s, st: (0, 0))],
        out_specs=pl.BlockSpec((npad, _K), lambda s, st: (0, 0)),
        scratch_shapes=[pltpu.VMEM((_P, _P), jnp.float32)],
    )
    return pl.pallas_call(
        functools.partial(_knn_body, P=_P, K=_K),
        grid_spec=grid_spec,
        out_shape=jax.ShapeDtypeStruct((npad, _K), jnp.int32),
    )(starts, xp)


def _sc_gather(table, idx, *, nb, D, CH):
    NW = 32
    bpw = nb // NW
    mesh = plsc.VectorSubcoreMesh(core_axis_name="c", subcore_axis_name="s")

    nch = bpw // CH

    @functools.partial(
        pl.kernel, mesh=mesh,
        out_type=jax.ShapeDtypeStruct((nb, D), jnp.float32),
        scratch_types=[pltpu.VMEM((bpw,), jnp.int32),
                       pltpu.VMEM((CH, D), jnp.float32),
                       pltpu.VMEM((CH, D), jnp.float32),
                       pltpu.SemaphoreType.DMA,
                       pltpu.SemaphoreType.DMA],
        compiler_params=pltpu.CompilerParams(use_tc_tiling_on_sc=False),
    )
    def gk(table_hbm, idx_hbm, out_hbm, idx_v, rows0, rows1, sem0, sem1):
        wid = lax.axis_index("s") * 2 + lax.axis_index("c")
        base = wid * bpw
        pltpu.sync_copy(idx_hbm.at[pl.ds(base, bpw)], idx_v)
        bufs, sems = (rows0, rows1), (sem0, sem1)
        cps = [pltpu.async_copy(
            table_hbm.at[idx_v.at[pl.ds(0, CH)]], rows0, sem0)]
        for c in range(nch):
            if c + 1 < nch:
                cps.append(pltpu.async_copy(
                    table_hbm.at[idx_v.at[pl.ds((c + 1) * CH, CH)]],
                    bufs[(c + 1) % 2], sems[(c + 1) % 2]))
            cps[c].wait()
            pltpu.sync_copy(bufs[c % 2], out_hbm.at[pl.ds(base + c * CH, CH)])

    return gk(table, idx)


def _edge1_body(u_ref, g_ref, w1_ref, b1_ref, rm1_ref, q1_ref, ga1_ref,
                be1_ref, w2_ref, b2_ref, rm2_ref, q2_ref, ga2_ref, be2_ref,
                w3_ref, b3_ref, x1_ref, *, R, K):
    u = u_ref[...]
    m = jnp.concatenate(
        [u + g_ref[:, k * 8:(k + 1) * 8] for k in range(K)], axis=0)
    z = _dot(m, w1_ref[...]) + b1_ref[...]
    h = jnp.maximum((z - rm1_ref[...]) / q1_ref[...] * ga1_ref[...]
                    + be1_ref[...], 0.0)
    z = _dot(h, w2_ref[...]) + b2_ref[...]
    h = jnp.maximum((z - rm2_ref[...]) / q2_ref[...] * ga2_ref[...]
                    + be2_ref[...], 0.0)
    e = _dot(h, w3_ref[...]) + b3_ref[...]
    x1 = e[0:R]
    for k in range(1, K):
        x1 = jnp.maximum(x1, e[k * R:(k + 1) * R])
    x1_ref[...] = x1


def _edge1(u1, gj, params, *, R, npad):
    n = u1.shape[0]
    row = lambda v: v[None, :]
    (w1p, b1, rm1, q1, ga1, be1, w2, b2, rm2, q2, ga2, be2, w3, b3) = params
    ins = (u1, gj, w1p, row(b1), row(rm1), row(q1), row(ga1), row(be1),
           w2, row(b2), row(rm2), row(q2), row(ga2), row(be2), w3, row(b3))
    specs = [pl.BlockSpec((R, 8), lambda i: (i, 0)),
             pl.BlockSpec((R, _K * 8), lambda i: (i, 0)),
             pl.BlockSpec((8, 64), lambda i: (0, 0))]
    specs += [pl.BlockSpec((1, 64), lambda i: (0, 0))] * 5
    specs += [pl.BlockSpec((64, 64), lambda i: (0, 0))]
    specs += [pl.BlockSpec((1, 64), lambda i: (0, 0))] * 5
    specs += [pl.BlockSpec((64, 64), lambda i: (0, 0)),
              pl.BlockSpec((1, 64), lambda i: (0, 0))]
    return pl.pallas_call(
        functools.partial(_edge1_body, R=R, K=_K),
        grid=(n // R,),
        in_specs=specs,
        out_specs=pl.BlockSpec((R, 64), lambda i: (i, 0)),
        out_shape=jax.ShapeDtypeStruct((npad, 64), jnp.float32),
    )(*ins)


def _pool_body(starts_ref, x1_ref, g2_ref, w4_ref, b4_ref,
               wl_ref, bl_ref, out_ref, *, R, K, B):
    pid = pl.program_id(0)
    x1 = x1_ref[...]
    m2 = jnp.concatenate(
        [jnp.concatenate([x1, g2_ref[:, k * 64:(k + 1) * 64] - x1], axis=1)
         for k in range(K)], axis=0)
    e = _dot(m2, w4_ref[...]) + b4_ref[...]
    x2 = e[0:R]
    for k in range(1, K):
        x2 = jnp.maximum(x2, e[k * R:(k + 1) * R])
    feat = jnp.concatenate([x1, x2], axis=1)
    lin = _dot(feat, wl_ref[...]) + bl_ref[...]
    base = pid * R
    rows = base + lax.broadcasted_iota(jnp.int32, (R, 1), 0)
    NEG = jnp.float32(-3e38)

    @pl.when(pid == 0)
    def _():
        out_ref[...] = jnp.full(out_ref.shape, NEG, jnp.float32)

    s_lo = jnp.int32(0)
    s_hi = jnp.int32(0)
    for s in range(B):
        s_lo += jnp.where(starts_ref[s + 1] <= base, 1, 0).astype(jnp.int32)
        s_hi += jnp.where(starts_ref[s] < base + R, 1, 0).astype(jnp.int32)

    def seg_body(s, _):
        msk = (rows >= starts_ref[s]) & (rows < starts_ref[s + 1])
        part = jnp.max(jnp.where(msk, lin, NEG), axis=0, keepdims=True)
        out_ref[pl.ds(s, 1), :] = jnp.maximum(out_ref[pl.ds(s, 1), :], part)
        return 0

    lax.fori_loop(s_lo, s_hi, seg_body, 0)


def _pool(x1, g2v, w4, b4, wl, bl, starts, *, R, n):
    grid_spec = pltpu.PrefetchScalarGridSpec(
        num_scalar_prefetch=1,
        grid=(n // R,),
        in_specs=[pl.BlockSpec((R, 64), lambda i, st: (i, 0)),
                  pl.BlockSpec((R, _K * 64), lambda i, st: (i, 0)),
                  pl.BlockSpec((128, 128), lambda i, st: (0, 0)),
                  pl.BlockSpec((1, 128), lambda i, st: (0, 0)),
                  pl.BlockSpec((192, 1024), lambda i, st: (0, 0)),
                  pl.BlockSpec((1, 1024), lambda i, st: (0, 0))],
        out_specs=pl.BlockSpec((_B, 1024), lambda i, st: (0, 0)),
    )
    return pl.pallas_call(
        functools.partial(_pool_body, R=R, K=_K, B=_B),
        grid_spec=grid_spec,
        out_shape=jax.ShapeDtypeStruct((_B, 1024), jnp.float32),
    )(starts, x1, g2v, w4, b4[None, :], wl, bl[None, :])


def _head_body(p_ref, w1_ref, b1_ref, w2_ref, b2_ref, w3_ref, b3_ref, o_ref):
    h = jnp.maximum(_dot(p_ref[...], w1_ref[...]) + b1_ref[...], 0.0)
    h = jnp.maximum(_dot(h, w2_ref[...]) + b2_ref[...], 0.0)
    o_ref[...] = _dot(h, w3_ref[...]) + b3_ref[...]


def _head(pooled, wm1, bm1, wm2, bm2, wm3, bm3):
    return pl.pallas_call(
        _head_body,
        out_shape=jax.ShapeDtypeStruct((_B, 40), jnp.float32),
    )(pooled, wm1, bm1[None, :], wm2, bm2[None, :], wm3, bm3[None, :])


def kernel(pos, batch, w1, b1, g1, be1, rm1, rv1, w2, b2, g2, be2, rm2, rv2,
           w3, b3, w4, b4, wl, bl, wm1, bm1, wm2, bm2, wm3, bm3):
    n = pos.shape[0]
    npad = n + _P
    starts = jnp.searchsorted(
        batch, jnp.arange(_B + 1, dtype=jnp.int32), side='left').astype(jnp.int32)

    q1 = jnp.sqrt(rv1 + 1e-5)
    q2 = jnp.sqrt(rv2 + 1e-5)
    w1p = jnp.pad(w1, ((0, 2), (0, 0)))
    zc = jnp.zeros((n, 2), jnp.float32)
    u1 = jnp.concatenate([pos, -pos, zc], axis=1)
    tbl1 = jnp.pad(pos, ((0, 0), (3, 2)))

    posp = jnp.pad(pos, ((0, npad - n), (0, 5)))
    idx1 = _knn(posp, starts, F=8)
    gj = _sc_gather(tbl1, idx1.reshape(-1), nb=n * _K, D=8, CH=1280)
    x1p = _edge1(
        u1, gj.reshape(n, _K * 8),
        (w1p, b1, rm1, q1, g1, be1, w2, b2, rm2, q2, g2, be2, w3, b3),
        R=512, npad=npad)

    idx2 = _knn(x1p, starts, F=64)
    g2r = _sc_gather(x1p, idx2.reshape(-1), nb=n * _K, D=64, CH=640)

    pooled = _pool(x1p, g2r.reshape(n, _K * 64), w4, b4, wl, bl,
                   starts, R=256, n=n)
    return _head(pooled, wm1, bm1, wm2, bm2, wm3, bm3)


# speedup vs baseline: 31.3831x; 1.0034x over previous
import functools

import jax
import jax.numpy as jnp
from jax import lax
from jax.experimental import pallas as pl
from jax.experimental.pallas import tpu as pltpu
from jax.experimental.pallas import tpu_sc as plsc

_B = 16
_K = 20
_P = 640


def _dot(a, b):
    return lax.dot_general(a, b, (((1,), (0,)), ((), ())),
                           preferred_element_type=jnp.float32)


def _knn_body(starts_ref, x_ref, idx_ref, d_ref, *, P, K):
    s = pl.program_id(0)
    st = starts_ref[s]
    size = starts_ref[s + 1] - st
    x = x_ref[pl.ds(st, P), :]
    sq = jnp.sum(x * x, axis=1, keepdims=True)
    g = lax.dot_general(x, x, (((1,), (1,)), ((), ())),
                        preferred_element_type=jnp.float32)
    d2 = (sq - 2.0 * g) + jnp.transpose(sq)
    col = lax.broadcasted_iota(jnp.int32, (1, P), 1)
    BIG = jnp.float32(3e38)
    d_ref[...] = jnp.where(col < size, d2, BIG)
    kcol = lax.broadcasted_iota(jnp.int32, (1, K), 1)

    def body(k, loc):
        d = d_ref[...]
        m = jnp.min(d, axis=1, keepdims=True)
        a = jnp.min(jnp.where(d <= m, col, P), axis=1, keepdims=True)
        d_ref[...] = jnp.where(col == a, BIG, d)
        return jnp.where(kcol == k, a, loc)

    loc = lax.fori_loop(0, K, body, jnp.zeros((P, K), jnp.int32))
    idx_ref[pl.ds(st, P), :] = loc + st


def _knn(xp, starts, *, F):
    npad = xp.shape[0]
    grid_spec = pltpu.PrefetchScalarGridSpec(
        num_scalar_prefetch=1,
        grid=(_B,),
        in_specs=[pl.BlockSpec((npad, F), lambda s, st: (0, 0))],
        out_specs=pl.BlockSpec((npad, _K), lambda s, st: (0, 0)),
        scratch_shapes=[pltpu.VMEM((_P, _P), jnp.float32)],
    )
    return pl.pallas_call(
        functools.partial(_knn_body, P=_P, K=_K),
        grid_spec=grid_spec,
        out_shape=jax.ShapeDtypeStruct((npad, _K), jnp.int32),
    )(starts, xp)


def _sc_gather(table, idx, *, nb, D, CH):
    NW = 32
    bpw = nb // NW
    mesh = plsc.VectorSubcoreMesh(core_axis_name="c", subcore_axis_name="s")

    nch = bpw // CH

    @functools.partial(
        pl.kernel, mesh=mesh,
        out_type=jax.ShapeDtypeStruct((nb, D), jnp.float32),
        scratch_types=[pltpu.VMEM((bpw,), jnp.int32),
                       pltpu.VMEM((CH, D), jnp.float32),
                       pltpu.VMEM((CH, D), jnp.float32),
                       pltpu.SemaphoreType.DMA,
                       pltpu.SemaphoreType.DMA],
        compiler_params=pltpu.CompilerParams(use_tc_tiling_on_sc=False),
    )
    def gk(table_hbm, idx_hbm, out_hbm, idx_v, rows0, rows1, sem0, sem1):
        wid = lax.axis_index("s") * 2 + lax.axis_index("c")
        base = wid * bpw
        pltpu.sync_copy(idx_hbm.at[pl.ds(base, bpw)], idx_v)
        bufs, sems = (rows0, rows1), (sem0, sem1)
        cps = [pltpu.async_copy(
            table_hbm.at[idx_v.at[pl.ds(0, CH)]], rows0, sem0)]
        for c in range(nch):
            if c + 1 < nch:
                cps.append(pltpu.async_copy(
                    table_hbm.at[idx_v.at[pl.ds((c + 1) * CH, CH)]],
                    bufs[(c + 1) % 2], sems[(c + 1) % 2]))
            cps[c].wait()
            pltpu.sync_copy(bufs[c % 2], out_hbm.at[pl.ds(base + c * CH, CH)])

    return gk(table, idx)


def _edge1_body(u_ref, g_ref, w1_ref, b1_ref, rm1_ref, q1_ref, ga1_ref,
                be1_ref, w2_ref, b2_ref, rm2_ref, q2_ref, ga2_ref, be2_ref,
                w3_ref, b3_ref, x1_ref, *, R, K):
    u = u_ref[...]
    m = jnp.concatenate(
        [u + g_ref[:, k * 8:(k + 1) * 8] for k in range(K)], axis=0)
    z = _dot(m, w1_ref[...]) + b1_ref[...]
    h = jnp.maximum((z - rm1_ref[...]) / q1_ref[...] * ga1_ref[...]
                    + be1_ref[...], 0.0)
    z = _dot(h, w2_ref[...]) + b2_ref[...]
    h = jnp.maximum((z - rm2_ref[...]) / q2_ref[...] * ga2_ref[...]
                    + be2_ref[...], 0.0)
    e = _dot(h, w3_ref[...]) + b3_ref[...]
    x1 = e[0:R]
    for k in range(1, K):
        x1 = jnp.maximum(x1, e[k * R:(k + 1) * R])
    x1_ref[...] = x1


def _edge1(u1, gj, params, *, R, npad):
    n = u1.shape[0]
    row = lambda v: v[None, :]
    (w1p, b1, rm1, q1, ga1, be1, w2, b2, rm2, q2, ga2, be2, w3, b3) = params
    ins = (u1, gj, w1p, row(b1), row(rm1), row(q1), row(ga1), row(be1),
           w2, row(b2), row(rm2), row(q2), row(ga2), row(be2), w3, row(b3))
    specs = [pl.BlockSpec((R, 8), lambda i: (i, 0)),
             pl.BlockSpec((R, _K * 8), lambda i: (i, 0)),
             pl.BlockSpec((8, 64), lambda i: (0, 0))]
    specs += [pl.BlockSpec((1, 64), lambda i: (0, 0))] * 5
    specs += [pl.BlockSpec((64, 64), lambda i: (0, 0))]
    specs += [pl.BlockSpec((1, 64), lambda i: (0, 0))] * 5
    specs += [pl.BlockSpec((64, 64), lambda i: (0, 0)),
              pl.BlockSpec((1, 64), lambda i: (0, 0))]
    return pl.pallas_call(
        functools.partial(_edge1_body, R=R, K=_K),
        grid=(n // R,),
        in_specs=specs,
        out_specs=pl.BlockSpec((R, 64), lambda i: (i, 0)),
        out_shape=jax.ShapeDtypeStruct((npad, 64), jnp.float32),
    )(*ins)


def _pool_body(starts_ref, x1_ref, g2_ref, w4_ref, b4_ref, wl_ref, bl_ref,
               wm1_ref, bm1_ref, wm2_ref, bm2_ref, wm3_ref, bm3_ref,
               out_ref, pool_scr, *, R, K, B, NB):
    pid = pl.program_id(0)
    NEG = jnp.float32(-3e38)

    @pl.when(pid == 0)
    def _():
        pool_scr[...] = jnp.full(pool_scr.shape, NEG, jnp.float32)

    @pl.when(pid < NB)
    def _():
        x1 = x1_ref[...]
        m2 = jnp.concatenate(
            [jnp.concatenate([x1, g2_ref[:, k * 64:(k + 1) * 64] - x1],
                             axis=1) for k in range(K)], axis=0)
        e = _dot(m2, w4_ref[...]) + b4_ref[...]
        x2 = e[0:R]
        for k in range(1, K):
            x2 = jnp.maximum(x2, e[k * R:(k + 1) * R])
        feat = jnp.concatenate([x1, x2], axis=1)
        lin = _dot(feat, wl_ref[...]) + bl_ref[...]
        base = pid * R
        rows = base + lax.broadcasted_iota(jnp.int32, (R, 1), 0)

        s_lo = jnp.int32(0)
        s_hi = jnp.int32(0)
        for s in range(B):
            s_lo += jnp.where(starts_ref[s + 1] <= base, 1, 0).astype(jnp.int32)
            s_hi += jnp.where(starts_ref[s] < base + R, 1, 0).astype(jnp.int32)

        def seg_body(s, _):
            msk = (rows >= starts_ref[s]) & (rows < starts_ref[s + 1])
            part = jnp.max(jnp.where(msk, lin, NEG), axis=0, keepdims=True)
            pool_scr[pl.ds(s, 1), :] = jnp.maximum(
                pool_scr[pl.ds(s, 1), :], part)
            return 0

        lax.fori_loop(s_lo, s_hi, seg_body, 0)

    @pl.when(pid == NB)
    def _():
        h = jnp.maximum(_dot(pool_scr[...], wm1_ref[...]) + bm1_ref[...], 0.0)
        h = jnp.maximum(_dot(h, wm2_ref[...]) + bm2_ref[...], 0.0)
        out_ref[...] = _dot(h, wm3_ref[...]) + bm3_ref[...]


def _pool(x1, g2v, w4, b4, wl, bl, wm1, bm1, wm2, bm2, wm3, bm3,
          starts, *, R, n):
    nb = n // R
    last = nb - 1
    blk = lambda i, st: (jnp.minimum(i, last), 0)
    zero = lambda i, st: (0, 0)
    grid_spec = pltpu.PrefetchScalarGridSpec(
        num_scalar_prefetch=1,
        grid=(nb + 1,),
        in_specs=[pl.BlockSpec((R, 64), blk),
                  pl.BlockSpec((R, _K * 64), blk),
                  pl.BlockSpec((128, 128), zero),
                  pl.BlockSpec((1, 128), zero),
                  pl.BlockSpec((192, 1024), zero),
                  pl.BlockSpec((1, 1024), zero),
                  pl.BlockSpec((1024, 512), zero),
                  pl.BlockSpec((1, 512), zero),
                  pl.BlockSpec((512, 256), zero),
                  pl.BlockSpec((1, 256), zero),
                  pl.BlockSpec((256, 40), zero),
                  pl.BlockSpec((1, 40), zero)],
        out_specs=pl.BlockSpec((_B, 40), zero),
        scratch_shapes=[pltpu.VMEM((_B, 1024), jnp.float32)],
    )
    return pl.pallas_call(
        functools.partial(_pool_body, R=R, K=_K, B=_B, NB=nb),
        grid_spec=grid_spec,
        out_shape=jax.ShapeDtypeStruct((_B, 40), jnp.float32),
    )(starts, x1, g2v, w4, b4[None, :], wl, bl[None, :],
      wm1, bm1[None, :], wm2, bm2[None, :], wm3, bm3[None, :])


def kernel(pos, batch, w1, b1, g1, be1, rm1, rv1, w2, b2, g2, be2, rm2, rv2,
           w3, b3, w4, b4, wl, bl, wm1, bm1, wm2, bm2, wm3, bm3):
    n = pos.shape[0]
    npad = n + _P
    starts = jnp.searchsorted(
        batch, jnp.arange(_B + 1, dtype=jnp.int32), side='left').astype(jnp.int32)

    q1 = jnp.sqrt(rv1 + 1e-5)
    q2 = jnp.sqrt(rv2 + 1e-5)
    w1p = jnp.pad(w1, ((0, 2), (0, 0)))
    zc = jnp.zeros((n, 2), jnp.float32)
    u1 = jnp.concatenate([pos, -pos, zc], axis=1)
    tbl1 = jnp.pad(pos, ((0, 0), (3, 2)))

    posp = jnp.pad(pos, ((0, npad - n), (0, 5)))
    idx1 = _knn(posp, starts, F=8)
    gj = _sc_gather(tbl1, idx1.reshape(-1), nb=n * _K, D=8, CH=1280)
    x1p = _edge1(
        u1, gj.reshape(n, _K * 8),
        (w1p, b1, rm1, q1, g1, be1, w2, b2, rm2, q2, g2, be2, w3, b3),
        R=512, npad=npad)

    idx2 = _knn(x1p, starts, F=64)
    g2r = _sc_gather(x1p, idx2.reshape(-1), nb=n * _K, D=64, CH=640)

    return _pool(x1p, g2r.reshape(n, _K * 64), w4, b4, wl, bl,
                 wm1, bm1, wm2, bm2, wm3, bm3, starts, R=256, n=n)
